# baseline (speedup 1.0000x reference)
import functools

import jax
import jax.numpy as jnp
import numpy as np
from jax import lax
from jax.experimental import pallas as pl
from jax.experimental.pallas import tpu as pltpu
from jax.experimental.pallas import tpu_sc as plsc

D_MODEL = 256
N_LEVELS = 4
N_HEADS = 8
N_POINTS = 4
DH = D_MODEL // N_HEADS
SPATIAL = [(64, 64), (32, 32), (16, 16), (8, 8)]
LVL_START = [0, 4096, 5120, 5376]
LIN = 5440
B = 2
LQ = LIN

NC = 2
NS = 16
NW = NC * NS
QPW = (B * LQ) // NW
QB = 2
NBLK = QPW // QB
SPQ = N_HEADS * N_LEVELS * N_POINTS
ROWS_PER_BLK = QB * SPQ

_TCBLK = 1088
_GRID = (B * LQ) // _TCBLK


def _lane_tables():
    j = np.arange(128)
    l = (j >> 2) & 3
    h = j >> 4
    w = np.array([64, 32, 16, 8], np.float32)[l]
    ls = np.array(LVL_START, np.float32)[l]
    sx = np.zeros((16, 128), np.float32)
    sy = np.zeros((16, 128), np.float32)
    sx[2 * l, j] = 1.0
    sy[2 * l + 1, j] = 1.0
    seg = np.zeros((128, 128), np.float32)
    seg[(j[:, None] >> 4) == (j[None, :] >> 4)] = 1.0
    return (jnp.asarray(w)[None], jnp.asarray(ls)[None],
            jnp.asarray(h.astype(np.float32))[None],
            jnp.asarray(sx), jnp.asarray(sy), jnp.asarray(seg))


def _prep_body(q_ref, f_ref, rp_ref, sowx_ref, sowy_ref, aww_ref, vpw_ref,
               sobx_ref, soby_ref, awb_ref, vpb_ref,
               wl_ref, ls_ref, hh_ref, sx_ref, sy_ref, seg_ref,
               idx_ref, w4_ref, val_ref):
    q = q_ref[...]
    hp = jax.lax.Precision.HIGHEST
    offx = jnp.dot(q, sowx_ref[...], preferred_element_type=jnp.float32) + sobx_ref[...]
    offy = jnp.dot(q, sowy_ref[...], preferred_element_type=jnp.float32) + soby_ref[...]
    logits = jnp.dot(q, aww_ref[...], preferred_element_type=jnp.float32) + awb_ref[...]
    val_ref[...] = jnp.dot(f_ref[...], vpw_ref[...], preferred_element_type=jnp.float32) + vpb_ref[...]
    rp = rp_ref[...]
    rpx = jnp.dot(rp, sx_ref[...], precision=hp, preferred_element_type=jnp.float32)
    rpy = jnp.dot(rp, sy_ref[...], precision=hp, preferred_element_type=jnp.float32)
    m = jnp.max(logits, axis=1, keepdims=True)
    e = jnp.exp(logits - m)
    s = jnp.dot(e, seg_ref[...], precision=hp, preferred_element_type=jnp.float32)
    aw = e / s
    wl = wl_ref[...]
    x = (rpx + offx / wl) * wl - 0.5
    y = (rpy + offy / wl) * wl - 0.5
    x0 = jnp.floor(x)
    fx = x - x0
    y0 = jnp.floor(y)
    fy = y - y0
    xs = jnp.clip(x0, 0.0, wl - 2.0)
    ys = jnp.clip(y0, 0.0, wl - 2.0)
    dx = x0 - xs
    dy = y0 - ys
    z = jnp.zeros_like(x)
    wxlo = jnp.where(dx == 0.0, 1.0 - fx, jnp.where(dx == -1.0, fx, z))
    wxhi = jnp.where(dx == 1.0, 1.0 - fx, jnp.where(dx == 0.0, fx, z))
    wylo = jnp.where(dy == 0.0, 1.0 - fy, jnp.where(dy == -1.0, fy, z)) * aw
    wyhi = jnp.where(dy == 1.0, 1.0 - fy, jnp.where(dy == 0.0, fy, z)) * aw
    w4_ref[:, 0:128] = wylo * wxlo
    w4_ref[:, 128:256] = wylo * wxhi
    w4_ref[:, 256:384] = wyhi * wxlo
    w4_ref[:, 384:512] = wyhi * wxhi
    bsel = (pl.program_id(0) >= _GRID // 2).astype(jnp.float32)
    gidx = (bsel * LIN + ls_ref[...] + ys * wl + xs) * N_HEADS + hh_ref[...]
    idx_ref[...] = gidx.astype(jnp.int32)


def _prep_tc(q2, f2, rp, sowx, sowy, aww, vpw, sobx, soby, awb, vpb):
    n = q2.shape[0]
    blk = _TCBLK
    wl, ls, hh, sx, sy, seg = _lane_tables()
    rep = lambda shp: pl.BlockSpec(shp, lambda i: tuple(0 for _ in shp))
    return pl.pallas_call(
        _prep_body,
        grid=(n // blk,),
        in_specs=[
            pl.BlockSpec((blk, D_MODEL), lambda i: (i, 0)),
            pl.BlockSpec((blk, D_MODEL), lambda i: (i, 0)),
            pl.BlockSpec((blk, 16), lambda i: (i, 0)),
            rep((D_MODEL, 128)), rep((D_MODEL, 128)), rep((D_MODEL, 128)),
            rep((D_MODEL, 256)),
            rep((128,)), rep((128,)), rep((128,)), rep((256,)),
            rep((1, 128)), rep((1, 128)), rep((1, 128)),
            rep((16, 128)), rep((16, 128)), rep((128, 128)),
        ],
        out_specs=[
            pl.BlockSpec((blk, 128), lambda i: (i, 0)),
            pl.BlockSpec((blk, 512), lambda i: (i, 0)),
            pl.BlockSpec((blk, 256), lambda i: (i, 0)),
        ],
        out_shape=[
            jax.ShapeDtypeStruct((n, 128), jnp.int32),
            jax.ShapeDtypeStruct((n, 512), jnp.float32),
            jax.ShapeDtypeStruct((n, 256), jnp.float32),
        ],
    )(q2, f2, rp, sowx, sowy, aww, vpw, sobx, soby, awb, vpb,
      wl, ls, hh, sx, sy, seg)


def _proj_body(x_ref, w_ref, b_ref, o_ref):
    o_ref[...] = jnp.dot(x_ref[...], w_ref[...], preferred_element_type=jnp.float32) + b_ref[...]


def _proj(x, w_t, b):
    n, k = x.shape
    m = w_t.shape[1]
    blk = _TCBLK
    return pl.pallas_call(
        _proj_body,
        grid=(n // blk,),
        in_specs=[
            pl.BlockSpec((blk, k), lambda i: (i, 0)),
            pl.BlockSpec((k, m), lambda i: (0, 0)),
            pl.BlockSpec((m,), lambda i: (0,)),
        ],
        out_specs=pl.BlockSpec((blk, m), lambda i: (i, 0)),
        out_shape=jax.ShapeDtypeStruct((n, m), jnp.float32),
    )(x, w_t, b)


def _sc_body(table_h, idx_h, w_h, out_h,
             out_v, idx_s0, idx_s1, w_s0, w_s1, rows_0, rows_1,
             sem_t0, sem_t1, sem_g0, sem_g1):
    iota = lax.iota(jnp.int32, 16)
    zero16 = jnp.full((16,), 0.0, jnp.float32)

    wid = lax.axis_index("c") * NS + lax.axis_index("s")
    q0 = wid * QPW

    def stage(k, idx_s, w_s, sem):
        qf = q0 + k * QB
        pltpu.async_copy(idx_h.at[pl.ds(qf, QB)], idx_s, sem)
        pltpu.async_copy(w_h.at[pl.ds(qf * 512, QB * 512)], w_s, sem)

    def wait_stage(k, idx_s, w_s, sem):
        qf = q0 + k * QB
        pltpu.make_async_copy(idx_h.at[pl.ds(qf, QB)], idx_s, sem).wait()
        pltpu.make_async_copy(w_h.at[pl.ds(qf * 512, QB * 512)], w_s, sem).wait()

    def fire(idx_s, rows_r, sem):
        for qq in range(QB):
            pltpu.async_copy(table_h.at[idx_s.at[qq]],
                             rows_r.at[pl.ds(qq * SPQ, SPQ)], sem)

    def drain(idx_s, rows_r, sem):
        for qq in range(QB):
            pltpu.make_async_copy(table_h.at[idx_s.at[qq]],
                                  rows_r.at[pl.ds(qq * SPQ, SPQ)], sem).wait()

    def accum(k, w_s, rows_r):
        qf = q0 + k * QB
        for qq in range(QB):
            for h in range(N_HEADS):
                rbase = qq * SPQ + h * 16
                wbase = qq * 512 + h * 16
                wsl = [w_s[pl.ds(wbase + slot * 128, 16)] for slot in range(4)]

                def body(t, carry, rbase=rbase, wsl=wsl):
                    lo, hi = carry
                    for u in range(4):
                        s = rbase + t * 4 + u
                        tv = jnp.full((16,), u, jnp.int32) + t * 4
                        for slot in range(4):
                            w = wsl[slot].at[tv].get(mode="promise_in_bounds")
                            lo = lo + w * rows_r[s, pl.ds(slot * 32, 16)]
                            hi = hi + w * rows_r[s, pl.ds(slot * 32 + 16, 16)]
                    return lo, hi

                lo, hi = lax.fori_loop(0, 4, body, (zero16, zero16))
                out_v[qq, pl.ds(h * 32, 16)] = lo
                out_v[qq, pl.ds(h * 32 + 16, 16)] = hi
        pltpu.sync_copy(out_v, out_h.at[pl.ds(qf, QB)])

    stage(0, idx_s0, w_s0, sem_t0)
    wait_stage(0, idx_s0, w_s0, sem_t0)
    fire(idx_s0, rows_0, sem_g0)
    stage(1, idx_s1, w_s1, sem_t1)

    def pipe(i, _):
        k = 2 * i
        wait_stage(k + 1, idx_s1, w_s1, sem_t1)
        fire(idx_s1, rows_1, sem_g1)
        drain(idx_s0, rows_0, sem_g0)
        accum(k, w_s0, rows_0)
        stage(k + 2, idx_s0, w_s0, sem_t0)
        drain(idx_s1, rows_1, sem_g1)
        accum(k + 1, w_s1, rows_1)
        stage(k + 3, idx_s1, w_s1, sem_t1)
        wait_stage(k + 2, idx_s0, w_s0, sem_t0)
        fire(idx_s0, rows_0, sem_g0)
        return 0

    lax.fori_loop(0, NBLK // 2 - 1, pipe, 0)
    wait_stage(NBLK - 1, idx_s1, w_s1, sem_t1)
    fire(idx_s1, rows_1, sem_g1)
    drain(idx_s0, rows_0, sem_g0)
    accum(NBLK - 2, w_s0, rows_0)
    drain(idx_s1, rows_1, sem_g1)
    accum(NBLK - 1, w_s1, rows_1)


@jax.jit
def _sc_gather(table, idx, w4, ):
    mesh = plsc.VectorSubcoreMesh(core_axis_name="c", subcore_axis_name="s",
                                  num_cores=NC, num_subcores=NS)
    f = functools.partial(
        pl.kernel,
        out_type=jax.ShapeDtypeStruct((B * LQ, D_MODEL), jnp.float32),
        mesh=mesh,
        scratch_types=[
            pltpu.VMEM((QB, 256), jnp.float32),
            pltpu.VMEM((QB, 128), jnp.int32),
            pltpu.VMEM((QB, 128), jnp.int32),
            pltpu.VMEM((QB * 512,), jnp.float32),
            pltpu.VMEM((QB * 512,), jnp.float32),
            pltpu.VMEM((ROWS_PER_BLK, 128), jnp.float32),
            pltpu.VMEM((ROWS_PER_BLK, 128), jnp.float32),
            pltpu.SemaphoreType.DMA,
            pltpu.SemaphoreType.DMA,
            pltpu.SemaphoreType.DMA,
            pltpu.SemaphoreType.DMA,
        ],
        compiler_params=pltpu.CompilerParams(needs_layout_passes=False),
    )(_sc_body)
    return f(table, idx, w4)


def _build_patch_table(val):
    v = val.reshape(B, LIN, D_MODEL)
    parts = []
    for (H, W), s in zip(SPATIAL, LVL_START):
        vl = v[:, s:s + H * W]
        z = lambda n: jnp.zeros((B, n, D_MODEL), jnp.float32)
        v1 = jnp.concatenate([vl[:, 1:], z(1)], 1)
        vW = jnp.concatenate([vl[:, W:], z(W)], 1)
        vW1 = jnp.concatenate([vl[:, W + 1:], z(W + 1)], 1)
        parts.append(jnp.stack([vl, v1, vW, vW1], 2))
    patch = jnp.concatenate(parts, 1)
    patch = patch.reshape(B, LIN, 4, N_HEADS, DH).transpose(0, 1, 3, 2, 4)
    return patch.reshape(B * LIN * N_HEADS, 4 * DH)


def kernel(query, reference_points, input_flatten, input_spatial_shapes,
           input_level_start_index, so_w, so_b, aw_w, aw_b, vp_w, vp_b,
           op_w, op_b):
    q2 = query.reshape(B * LQ, D_MODEL)
    f2 = input_flatten.reshape(B * LIN, D_MODEL)
    rp = reference_points.reshape(B * LQ, 8)
    rp = jnp.concatenate([rp, jnp.zeros((B * LQ, 8), jnp.float32)], axis=1)
    sowt = so_w.T
    idx, w4, val = _prep_tc(q2, f2, rp, sowt[:, 0::2], sowt[:, 1::2], aw_w.T,
                            vp_w.T, so_b[0::2], so_b[1::2], aw_b, vp_b)
    table = _build_patch_table(val)
    out_core = _sc_gather(table, idx, w4.reshape(-1))
    o = _proj(out_core, op_w.T, op_b)
    return o.reshape(B, LQ, D_MODEL)


# speedup vs baseline: 1.1799x; 1.1799x over previous
import functools

import jax
import jax.numpy as jnp
import numpy as np
from jax import lax
from jax.experimental import pallas as pl
from jax.experimental.pallas import tpu as pltpu
from jax.experimental.pallas import tpu_sc as plsc

D_MODEL = 256
N_LEVELS = 4
N_HEADS = 8
N_POINTS = 4
DH = D_MODEL // N_HEADS
SPATIAL = [(64, 64), (32, 32), (16, 16), (8, 8)]
LVL_START = [0, 4096, 5120, 5376]
LIN = 5440
B = 2
LQ = LIN

NC = 2
NS = 16
NW = NC * NS
QPW = (B * LQ) // NW
QB = 2
NBLK = QPW // QB
SPQ = N_HEADS * N_LEVELS * N_POINTS
ROWS_PER_BLK = QB * SPQ

_TCBLK = 1088
_GRID = (B * LQ) // _TCBLK


def _lane_tables():
    j = np.arange(128)
    l = (j >> 2) & 3
    h = j >> 4
    w = np.array([64, 32, 16, 8], np.float32)[l]
    ls = np.array(LVL_START, np.float32)[l]
    sx = np.zeros((16, 128), np.float32)
    sy = np.zeros((16, 128), np.float32)
    sx[2 * l, j] = 1.0
    sy[2 * l + 1, j] = 1.0
    seg = np.zeros((128, 128), np.float32)
    seg[(j[:, None] >> 4) == (j[None, :] >> 4)] = 1.0
    return (jnp.asarray(w)[None], jnp.asarray(ls)[None],
            jnp.asarray(h.astype(np.float32))[None],
            jnp.asarray(sx), jnp.asarray(sy), jnp.asarray(seg))


def _prep_body(q_ref, f_ref, rp_ref, sowx_ref, sowy_ref, aww_ref, vpw_ref,
               sobx_ref, soby_ref, awb_ref, vpb_ref,
               wl_ref, ls_ref, hh_ref, sx_ref, sy_ref, seg_ref,
               idx_ref, w4_ref, val_ref):
    q = q_ref[...]
    hp = jax.lax.Precision.HIGHEST
    offx = jnp.dot(q, sowx_ref[...], preferred_element_type=jnp.float32) + sobx_ref[...]
    offy = jnp.dot(q, sowy_ref[...], preferred_element_type=jnp.float32) + soby_ref[...]
    logits = jnp.dot(q, aww_ref[...], preferred_element_type=jnp.float32) + awb_ref[...]
    val_ref[...] = jnp.dot(f_ref[...], vpw_ref[...], preferred_element_type=jnp.float32) + vpb_ref[...]
    rp = rp_ref[...]
    rpx = jnp.dot(rp, sx_ref[...], precision=hp, preferred_element_type=jnp.float32)
    rpy = jnp.dot(rp, sy_ref[...], precision=hp, preferred_element_type=jnp.float32)
    m = jnp.max(logits, axis=1, keepdims=True)
    e = jnp.exp(logits - m)
    s = jnp.dot(e, seg_ref[...], precision=hp, preferred_element_type=jnp.float32)
    aw = e / s
    wl = wl_ref[...]
    x = (rpx + offx / wl) * wl - 0.5
    y = (rpy + offy / wl) * wl - 0.5
    x0 = jnp.floor(x)
    fx = x - x0
    y0 = jnp.floor(y)
    fy = y - y0
    xs = jnp.clip(x0, 0.0, wl - 2.0)
    ys = jnp.clip(y0, 0.0, wl - 2.0)
    dx = x0 - xs
    dy = y0 - ys
    z = jnp.zeros_like(x)
    wxlo = jnp.where(dx == 0.0, 1.0 - fx, jnp.where(dx == -1.0, fx, z))
    wxhi = jnp.where(dx == 1.0, 1.0 - fx, jnp.where(dx == 0.0, fx, z))
    wylo = jnp.where(dy == 0.0, 1.0 - fy, jnp.where(dy == -1.0, fy, z)) * aw
    wyhi = jnp.where(dy == 1.0, 1.0 - fy, jnp.where(dy == 0.0, fy, z)) * aw
    w4_ref[:, 0:128] = wylo * wxlo
    w4_ref[:, 128:256] = wylo * wxhi
    w4_ref[:, 256:384] = wyhi * wxlo
    w4_ref[:, 384:512] = wyhi * wxhi
    bsel = (pl.program_id(0) >= _GRID // 2).astype(jnp.float32)
    gidx = (bsel * LIN + ls_ref[...] + ys * wl + xs) * N_HEADS + hh_ref[...]
    idx_ref[...] = gidx.astype(jnp.int32)


def _prep_tc(q2, f2, rp, sowx, sowy, aww, vpw, sobx, soby, awb, vpb):
    n = q2.shape[0]
    blk = _TCBLK
    wl, ls, hh, sx, sy, seg = _lane_tables()
    rep = lambda shp: pl.BlockSpec(shp, lambda i: tuple(0 for _ in shp))
    return pl.pallas_call(
        _prep_body,
        grid=(n // blk,),
        in_specs=[
            pl.BlockSpec((blk, D_MODEL), lambda i: (i, 0)),
            pl.BlockSpec((blk, D_MODEL), lambda i: (i, 0)),
            pl.BlockSpec((blk, 16), lambda i: (i, 0)),
            rep((D_MODEL, 128)), rep((D_MODEL, 128)), rep((D_MODEL, 128)),
            rep((D_MODEL, 256)),
            rep((128,)), rep((128,)), rep((128,)), rep((256,)),
            rep((1, 128)), rep((1, 128)), rep((1, 128)),
            rep((16, 128)), rep((16, 128)), rep((128, 128)),
        ],
        out_specs=[
            pl.BlockSpec((blk, 128), lambda i: (i, 0)),
            pl.BlockSpec((blk, 512), lambda i: (i, 0)),
            pl.BlockSpec((blk, 256), lambda i: (i, 0)),
        ],
        out_shape=[
            jax.ShapeDtypeStruct((n, 128), jnp.int32),
            jax.ShapeDtypeStruct((n, 512), jnp.float32),
            jax.ShapeDtypeStruct((n, 256), jnp.float32),
        ],
    )(q2, f2, rp, sowx, sowy, aww, vpw, sobx, soby, awb, vpb,
      wl, ls, hh, sx, sy, seg)


def _proj_body(x_ref, w_ref, b_ref, o_ref):
    o_ref[...] = jnp.dot(x_ref[...], w_ref[...], preferred_element_type=jnp.float32) + b_ref[...]


def _proj(x, w_t, b):
    n, k = x.shape
    m = w_t.shape[1]
    blk = _TCBLK
    return pl.pallas_call(
        _proj_body,
        grid=(n // blk,),
        in_specs=[
            pl.BlockSpec((blk, k), lambda i: (i, 0)),
            pl.BlockSpec((k, m), lambda i: (0, 0)),
            pl.BlockSpec((m,), lambda i: (0,)),
        ],
        out_specs=pl.BlockSpec((blk, m), lambda i: (i, 0)),
        out_shape=jax.ShapeDtypeStruct((n, m), jnp.float32),
    )(x, w_t, b)


def _sc_body(table_h, idx_h, w_h, out_h,
             out_v, idx_s0, idx_s1, w_s0, w_s1, rows_0, rows_1,
             sem_t0, sem_t1, sem_g0, sem_g1):
    iota = lax.iota(jnp.int32, 16)
    zero16 = jnp.full((16,), 0.0, jnp.float32)

    wid = lax.axis_index("c") * NS + lax.axis_index("s")
    q0 = wid * QPW

    def stage(k, idx_s, w_s, sem):
        qf = q0 + k * QB
        pltpu.async_copy(idx_h.at[pl.ds(qf, QB)], idx_s, sem)
        pltpu.async_copy(w_h.at[pl.ds(qf * 512, QB * 512)], w_s, sem)

    def wait_stage(k, idx_s, w_s, sem):
        qf = q0 + k * QB
        pltpu.make_async_copy(idx_h.at[pl.ds(qf, QB)], idx_s, sem).wait()
        pltpu.make_async_copy(w_h.at[pl.ds(qf * 512, QB * 512)], w_s, sem).wait()

    def fire(idx_s, rows_r, sem):
        for qq in range(QB):
            pltpu.async_copy(table_h.at[idx_s.at[qq]],
                             rows_r.at[pl.ds(qq * SPQ, SPQ)], sem)

    def drain(idx_s, rows_r, sem):
        for qq in range(QB):
            pltpu.make_async_copy(table_h.at[idx_s.at[qq]],
                                  rows_r.at[pl.ds(qq * SPQ, SPQ)], sem).wait()

    def accum(k, w_s, rows_r):
        qf = q0 + k * QB
        for qq in range(QB):
            for h in range(N_HEADS):
                rbase = qq * SPQ + h * 16
                wbase = qq * 512 + h * 16
                wsl = [w_s[pl.ds(wbase + slot * 128, 16)] for slot in range(4)]

                def body(t, carry, rbase=rbase, wsl=wsl):
                    lo, hi = carry
                    s = rbase + t
                    tv = jnp.full((16,), 0, jnp.int32) + t
                    for slot in range(4):
                        w = wsl[slot].at[tv].get(mode="promise_in_bounds")
                        lo = lo + w * rows_r[s, pl.ds(slot * 32, 16)]
                        hi = hi + w * rows_r[s, pl.ds(slot * 32 + 16, 16)]
                    return lo, hi

                lo, hi = lax.fori_loop(0, 16, body, (zero16, zero16))
                out_v[qq, pl.ds(h * 32, 16)] = lo
                out_v[qq, pl.ds(h * 32 + 16, 16)] = hi
        pltpu.sync_copy(out_v, out_h.at[pl.ds(qf, QB)])

    stage(0, idx_s0, w_s0, sem_t0)
    wait_stage(0, idx_s0, w_s0, sem_t0)
    fire(idx_s0, rows_0, sem_g0)
    stage(1, idx_s1, w_s1, sem_t1)

    def pipe(i, _):
        k = 2 * i
        wait_stage(k + 1, idx_s1, w_s1, sem_t1)
        fire(idx_s1, rows_1, sem_g1)
        drain(idx_s0, rows_0, sem_g0)
        accum(k, w_s0, rows_0)
        stage(k + 2, idx_s0, w_s0, sem_t0)
        drain(idx_s1, rows_1, sem_g1)
        accum(k + 1, w_s1, rows_1)
        stage(k + 3, idx_s1, w_s1, sem_t1)
        wait_stage(k + 2, idx_s0, w_s0, sem_t0)
        fire(idx_s0, rows_0, sem_g0)
        return 0

    lax.fori_loop(0, NBLK // 2 - 1, pipe, 0)
    wait_stage(NBLK - 1, idx_s1, w_s1, sem_t1)
    fire(idx_s1, rows_1, sem_g1)
    drain(idx_s0, rows_0, sem_g0)
    accum(NBLK - 2, w_s0, rows_0)
    drain(idx_s1, rows_1, sem_g1)
    accum(NBLK - 1, w_s1, rows_1)


@jax.jit
def _sc_gather(table, idx, w4, ):
    mesh = plsc.VectorSubcoreMesh(core_axis_name="c", subcore_axis_name="s",
                                  num_cores=NC, num_subcores=NS)
    f = functools.partial(
        pl.kernel,
        out_type=jax.ShapeDtypeStruct((B * LQ, D_MODEL), jnp.float32),
        mesh=mesh,
        scratch_types=[
            pltpu.VMEM((QB, 256), jnp.float32),
            pltpu.VMEM((QB, 128), jnp.int32),
            pltpu.VMEM((QB, 128), jnp.int32),
            pltpu.VMEM((QB * 512,), jnp.float32),
            pltpu.VMEM((QB * 512,), jnp.float32),
            pltpu.VMEM((ROWS_PER_BLK, 128), jnp.float32),
            pltpu.VMEM((ROWS_PER_BLK, 128), jnp.float32),
            pltpu.SemaphoreType.DMA,
            pltpu.SemaphoreType.DMA,
            pltpu.SemaphoreType.DMA,
            pltpu.SemaphoreType.DMA,
        ],
        compiler_params=pltpu.CompilerParams(needs_layout_passes=False),
    )(_sc_body)
    return f(table, idx, w4)


def _build_patch_table(val):
    v = val.reshape(B, LIN, D_MODEL)
    parts = []
    for (H, W), s in zip(SPATIAL, LVL_START):
        vl = v[:, s:s + H * W]
        z = lambda n: jnp.zeros((B, n, D_MODEL), jnp.float32)
        v1 = jnp.concatenate([vl[:, 1:], z(1)], 1)
        vW = jnp.concatenate([vl[:, W:], z(W)], 1)
        vW1 = jnp.concatenate([vl[:, W + 1:], z(W + 1)], 1)
        parts.append(jnp.stack([vl, v1, vW, vW1], 2))
    patch = jnp.concatenate(parts, 1)
    patch = patch.reshape(B, LIN, 4, N_HEADS, DH).transpose(0, 1, 3, 2, 4)
    return patch.reshape(B * LIN * N_HEADS, 4 * DH)


def kernel(query, reference_points, input_flatten, input_spatial_shapes,
           input_level_start_index, so_w, so_b, aw_w, aw_b, vp_w, vp_b,
           op_w, op_b):
    q2 = query.reshape(B * LQ, D_MODEL)
    f2 = input_flatten.reshape(B * LIN, D_MODEL)
    rp = reference_points.reshape(B * LQ, 8)
    rp = jnp.concatenate([rp, jnp.zeros((B * LQ, 8), jnp.float32)], axis=1)
    sowt = so_w.T
    idx, w4, val = _prep_tc(q2, f2, rp, sowt[:, 0::2], sowt[:, 1::2], aw_w.T,
                            vp_w.T, so_b[0::2], so_b[1::2], aw_b, vp_b)
    table = _build_patch_table(val)
    out_core = _sc_gather(table, idx, w4.reshape(-1))
    o = _proj(out_core, op_w.T, op_b)
    return o.reshape(B, LQ, D_MODEL)


# speedup vs baseline: 1.1804x; 1.0004x over previous
import functools

import jax
import jax.numpy as jnp
import numpy as np
from jax import lax
from jax.experimental import pallas as pl
from jax.experimental.pallas import tpu as pltpu
from jax.experimental.pallas import tpu_sc as plsc

D_MODEL = 256
N_LEVELS = 4
N_HEADS = 8
N_POINTS = 4
DH = D_MODEL // N_HEADS
SPATIAL = [(64, 64), (32, 32), (16, 16), (8, 8)]
LVL_START = [0, 4096, 5120, 5376]
LIN = 5440
B = 2
LQ = LIN

NC = 2
NS = 16
NW = NC * NS
QPW = (B * LQ) // NW
QB = 2
NBLK = QPW // QB
SPQ = N_HEADS * N_LEVELS * N_POINTS
ROWS_PER_BLK = QB * SPQ

_TCBLK = 1088
_GRID = (B * LQ) // _TCBLK


def _lane_tables():
    j = np.arange(128)
    l = (j >> 2) & 3
    h = j >> 4
    w = np.array([64, 32, 16, 8], np.float32)[l]
    ls = np.array(LVL_START, np.float32)[l]
    sx = np.zeros((16, 128), np.float32)
    sy = np.zeros((16, 128), np.float32)
    sx[2 * l, j] = 1.0
    sy[2 * l + 1, j] = 1.0
    seg = np.zeros((128, 128), np.float32)
    seg[(j[:, None] >> 4) == (j[None, :] >> 4)] = 1.0
    return (jnp.asarray(w)[None], jnp.asarray(ls)[None],
            jnp.asarray(h.astype(np.float32))[None],
            jnp.asarray(sx), jnp.asarray(sy), jnp.asarray(seg))


def _prep_body(q_ref, f_ref, rp_ref, sowx_ref, sowy_ref, aww_ref, vpw_ref,
               sobx_ref, soby_ref, awb_ref, vpb_ref,
               wl_ref, ls_ref, hh_ref, sx_ref, sy_ref, seg_ref,
               idx_ref, w4_ref, val_ref):
    q = q_ref[...]
    hp = jax.lax.Precision.HIGHEST
    offx = jnp.dot(q, sowx_ref[...], preferred_element_type=jnp.float32) + sobx_ref[...]
    offy = jnp.dot(q, sowy_ref[...], preferred_element_type=jnp.float32) + soby_ref[...]
    logits = jnp.dot(q, aww_ref[...], preferred_element_type=jnp.float32) + awb_ref[...]
    val_ref[...] = jnp.dot(f_ref[...], vpw_ref[...], preferred_element_type=jnp.float32) + vpb_ref[...]
    rp = rp_ref[...]
    rpx = jnp.dot(rp, sx_ref[...], precision=hp, preferred_element_type=jnp.float32)
    rpy = jnp.dot(rp, sy_ref[...], precision=hp, preferred_element_type=jnp.float32)
    m = jnp.max(logits, axis=1, keepdims=True)
    e = jnp.exp(logits - m)
    s = jnp.dot(e, seg_ref[...], precision=hp, preferred_element_type=jnp.float32)
    aw = e / s
    wl = wl_ref[...]
    x = (rpx + offx / wl) * wl - 0.5
    y = (rpy + offy / wl) * wl - 0.5
    x0 = jnp.floor(x)
    fx = x - x0
    y0 = jnp.floor(y)
    fy = y - y0
    xs = jnp.clip(x0, 0.0, wl - 2.0)
    ys = jnp.clip(y0, 0.0, wl - 2.0)
    dx = x0 - xs
    dy = y0 - ys
    z = jnp.zeros_like(x)
    wxlo = jnp.where(dx == 0.0, 1.0 - fx, jnp.where(dx == -1.0, fx, z))
    wxhi = jnp.where(dx == 1.0, 1.0 - fx, jnp.where(dx == 0.0, fx, z))
    wylo = jnp.where(dy == 0.0, 1.0 - fy, jnp.where(dy == -1.0, fy, z)) * aw
    wyhi = jnp.where(dy == 1.0, 1.0 - fy, jnp.where(dy == 0.0, fy, z)) * aw
    w4_ref[:, 0:128] = wylo * wxlo
    w4_ref[:, 128:256] = wylo * wxhi
    w4_ref[:, 256:384] = wyhi * wxlo
    w4_ref[:, 384:512] = wyhi * wxhi
    bsel = (pl.program_id(0) >= _GRID // 2).astype(jnp.float32)
    gidx = (bsel * LIN + ls_ref[...] + ys * wl + xs) * N_HEADS + hh_ref[...]
    idx_ref[...] = gidx.astype(jnp.int32)


def _prep_tc(q2, f2, rp, sowx, sowy, aww, vpw, sobx, soby, awb, vpb):
    n = q2.shape[0]
    blk = _TCBLK
    wl, ls, hh, sx, sy, seg = _lane_tables()
    rep = lambda shp: pl.BlockSpec(shp, lambda i: tuple(0 for _ in shp))
    return pl.pallas_call(
        _prep_body,
        grid=(n // blk,),
        in_specs=[
            pl.BlockSpec((blk, D_MODEL), lambda i: (i, 0)),
            pl.BlockSpec((blk, D_MODEL), lambda i: (i, 0)),
            pl.BlockSpec((blk, 16), lambda i: (i, 0)),
            rep((D_MODEL, 128)), rep((D_MODEL, 128)), rep((D_MODEL, 128)),
            rep((D_MODEL, 256)),
            rep((128,)), rep((128,)), rep((128,)), rep((256,)),
            rep((1, 128)), rep((1, 128)), rep((1, 128)),
            rep((16, 128)), rep((16, 128)), rep((128, 128)),
        ],
        out_specs=[
            pl.BlockSpec((blk, 128), lambda i: (i, 0)),
            pl.BlockSpec((blk, 512), lambda i: (i, 0)),
            pl.BlockSpec((blk, 256), lambda i: (i, 0)),
        ],
        out_shape=[
            jax.ShapeDtypeStruct((n, 128), jnp.int32),
            jax.ShapeDtypeStruct((n, 512), jnp.float32),
            jax.ShapeDtypeStruct((n, 256), jnp.float32),
        ],
    )(q2, f2, rp, sowx, sowy, aww, vpw, sobx, soby, awb, vpb,
      wl, ls, hh, sx, sy, seg)


def _proj_body(x_ref, w_ref, b_ref, o_ref):
    o_ref[...] = jnp.dot(x_ref[...], w_ref[...], preferred_element_type=jnp.float32) + b_ref[...]


def _proj(x, w_t, b):
    n, k = x.shape
    m = w_t.shape[1]
    blk = _TCBLK
    return pl.pallas_call(
        _proj_body,
        grid=(n // blk,),
        in_specs=[
            pl.BlockSpec((blk, k), lambda i: (i, 0)),
            pl.BlockSpec((k, m), lambda i: (0, 0)),
            pl.BlockSpec((m,), lambda i: (0,)),
        ],
        out_specs=pl.BlockSpec((blk, m), lambda i: (i, 0)),
        out_shape=jax.ShapeDtypeStruct((n, m), jnp.float32),
    )(x, w_t, b)


def _sc_body(table_h, idx_h, w_h, out_h,
             out_v, idx_s0, idx_s1, w_s0, w_s1, rows_0, rows_1,
             sem_t0, sem_t1, sem_g0, sem_g1):
    iota = lax.iota(jnp.int32, 16)
    zero16 = jnp.full((16,), 0.0, jnp.float32)

    wid = lax.axis_index("c") * NS + lax.axis_index("s")
    q0 = wid * QPW

    def stage(k, idx_s, w_s, sem):
        qf = q0 + k * QB
        pltpu.async_copy(idx_h.at[pl.ds(qf, QB)], idx_s, sem)
        pltpu.async_copy(w_h.at[pl.ds(qf * 512, QB * 512)], w_s, sem)

    def wait_stage(k, idx_s, w_s, sem):
        qf = q0 + k * QB
        pltpu.make_async_copy(idx_h.at[pl.ds(qf, QB)], idx_s, sem).wait()
        pltpu.make_async_copy(w_h.at[pl.ds(qf * 512, QB * 512)], w_s, sem).wait()

    def fire(idx_s, rows_r, sem):
        for qq in range(QB):
            pltpu.async_copy(table_h.at[idx_s.at[qq]],
                             rows_r.at[pl.ds(qq * SPQ, SPQ)], sem)

    def drain(idx_s, rows_r, sem):
        for qq in range(QB):
            pltpu.make_async_copy(table_h.at[idx_s.at[qq]],
                                  rows_r.at[pl.ds(qq * SPQ, SPQ)], sem).wait()

    def accum(k, w_s, rows_r):
        qf = q0 + k * QB
        for qq in range(QB):
            for h in range(N_HEADS):
                rbase = qq * SPQ + h * 16
                wbase = qq * 512 + h * 16
                wsl = [w_s[pl.ds(wbase + slot * 128, 16)] for slot in range(4)]

                def body(t, carry, rbase=rbase, wsl=wsl):
                    acc = list(carry)
                    s = rbase + t
                    tv = jnp.full((16,), 0, jnp.int32) + t
                    for slot in range(4):
                        w = wsl[slot].at[tv].get(mode="promise_in_bounds")
                        acc[slot] = acc[slot] + w * rows_r[s, pl.ds(slot * 32, 16)]
                        acc[slot + 4] = acc[slot + 4] + w * rows_r[s, pl.ds(slot * 32 + 16, 16)]
                    return tuple(acc)

                acc = lax.fori_loop(0, 16, body, (zero16,) * 8)
                lo = (acc[0] + acc[1]) + (acc[2] + acc[3])
                hi = (acc[4] + acc[5]) + (acc[6] + acc[7])
                out_v[qq, pl.ds(h * 32, 16)] = lo
                out_v[qq, pl.ds(h * 32 + 16, 16)] = hi
        pltpu.sync_copy(out_v, out_h.at[pl.ds(qf, QB)])

    stage(0, idx_s0, w_s0, sem_t0)
    wait_stage(0, idx_s0, w_s0, sem_t0)
    fire(idx_s0, rows_0, sem_g0)
    stage(1, idx_s1, w_s1, sem_t1)

    def pipe(i, _):
        k = 2 * i
        wait_stage(k + 1, idx_s1, w_s1, sem_t1)
        fire(idx_s1, rows_1, sem_g1)
        drain(idx_s0, rows_0, sem_g0)
        accum(k, w_s0, rows_0)
        stage(k + 2, idx_s0, w_s0, sem_t0)
        drain(idx_s1, rows_1, sem_g1)
        accum(k + 1, w_s1, rows_1)
        stage(k + 3, idx_s1, w_s1, sem_t1)
        wait_stage(k + 2, idx_s0, w_s0, sem_t0)
        fire(idx_s0, rows_0, sem_g0)
        return 0

    lax.fori_loop(0, NBLK // 2 - 1, pipe, 0)
    wait_stage(NBLK - 1, idx_s1, w_s1, sem_t1)
    fire(idx_s1, rows_1, sem_g1)
    drain(idx_s0, rows_0, sem_g0)
    accum(NBLK - 2, w_s0, rows_0)
    drain(idx_s1, rows_1, sem_g1)
    accum(NBLK - 1, w_s1, rows_1)


@jax.jit
def _sc_gather(table, idx, w4, ):
    mesh = plsc.VectorSubcoreMesh(core_axis_name="c", subcore_axis_name="s",
                                  num_cores=NC, num_subcores=NS)
    f = functools.partial(
        pl.kernel,
        out_type=jax.ShapeDtypeStruct((B * LQ, D_MODEL), jnp.float32),
        mesh=mesh,
        scratch_types=[
            pltpu.VMEM((QB, 256), jnp.float32),
            pltpu.VMEM((QB, 128), jnp.int32),
            pltpu.VMEM((QB, 128), jnp.int32),
            pltpu.VMEM((QB * 512,), jnp.float32),
            pltpu.VMEM((QB * 512,), jnp.float32),
            pltpu.VMEM((ROWS_PER_BLK, 128), jnp.float32),
            pltpu.VMEM((ROWS_PER_BLK, 128), jnp.float32),
            pltpu.SemaphoreType.DMA,
            pltpu.SemaphoreType.DMA,
            pltpu.SemaphoreType.DMA,
            pltpu.SemaphoreType.DMA,
        ],
        compiler_params=pltpu.CompilerParams(needs_layout_passes=False),
    )(_sc_body)
    return f(table, idx, w4)


def _build_patch_table(val):
    v = val.reshape(B, LIN, D_MODEL)
    parts = []
    for (H, W), s in zip(SPATIAL, LVL_START):
        vl = v[:, s:s + H * W]
        z = lambda n: jnp.zeros((B, n, D_MODEL), jnp.float32)
        v1 = jnp.concatenate([vl[:, 1:], z(1)], 1)
        vW = jnp.concatenate([vl[:, W:], z(W)], 1)
        vW1 = jnp.concatenate([vl[:, W + 1:], z(W + 1)], 1)
        parts.append(jnp.stack([vl, v1, vW, vW1], 2))
    patch = jnp.concatenate(parts, 1)
    patch = patch.reshape(B, LIN, 4, N_HEADS, DH).transpose(0, 1, 3, 2, 4)
    return patch.reshape(B * LIN * N_HEADS, 4 * DH)


def kernel(query, reference_points, input_flatten, input_spatial_shapes,
           input_level_start_index, so_w, so_b, aw_w, aw_b, vp_w, vp_b,
           op_w, op_b):
    q2 = query.reshape(B * LQ, D_MODEL)
    f2 = input_flatten.reshape(B * LIN, D_MODEL)
    rp = reference_points.reshape(B * LQ, 8)
    rp = jnp.concatenate([rp, jnp.zeros((B * LQ, 8), jnp.float32)], axis=1)
    sowt = so_w.T
    idx, w4, val = _prep_tc(q2, f2, rp, sowt[:, 0::2], sowt[:, 1::2], aw_w.T,
                            vp_w.T, so_b[0::2], so_b[1::2], aw_b, vp_b)
    table = _build_patch_table(val)
    out_core = _sc_gather(table, idx, w4.reshape(-1))
    o = _proj(out_core, op_w.T, op_b)
    return o.reshape(B, LQ, D_MODEL)


# speedup vs baseline: 1.1849x; 1.0038x over previous
import functools

import jax
import jax.numpy as jnp
import numpy as np
from jax import lax
from jax.experimental import pallas as pl
from jax.experimental.pallas import tpu as pltpu
from jax.experimental.pallas import tpu_sc as plsc

D_MODEL = 256
N_LEVELS = 4
N_HEADS = 8
N_POINTS = 4
DH = D_MODEL // N_HEADS
SPATIAL = [(64, 64), (32, 32), (16, 16), (8, 8)]
LVL_START = [0, 4096, 5120, 5376]
LIN = 5440
B = 2
LQ = LIN

NC = 2
NS = 16
NW = NC * NS
QPW = (B * LQ) // NW
QB = 2
NBLK = QPW // QB
SPQ = N_HEADS * N_LEVELS * N_POINTS
ROWS_PER_BLK = QB * SPQ

_TCBLK = 1088
_GRID = (B * LQ) // _TCBLK


def _lane_tables():
    j = np.arange(128)
    l = (j >> 2) & 3
    h = j >> 4
    w = np.array([64, 32, 16, 8], np.float32)[l]
    ls = np.array(LVL_START, np.float32)[l]
    sx = np.zeros((16, 128), np.float32)
    sy = np.zeros((16, 128), np.float32)
    sx[2 * l, j] = 1.0
    sy[2 * l + 1, j] = 1.0
    seg = np.zeros((128, 128), np.float32)
    seg[(j[:, None] >> 4) == (j[None, :] >> 4)] = 1.0
    return (jnp.asarray(w)[None], jnp.asarray(ls)[None],
            jnp.asarray(h.astype(np.float32))[None],
            jnp.asarray(sx), jnp.asarray(sy), jnp.asarray(seg))


def _prep_body(q_ref, f_ref, rp_ref, sowx_ref, sowy_ref, aww_ref, vpw_ref,
               sobx_ref, soby_ref, awb_ref, vpb_ref,
               wl_ref, ls_ref, hh_ref, sx_ref, sy_ref, seg_ref,
               idx_ref, w4_ref, val_ref):
    q = q_ref[...]
    hp = jax.lax.Precision.HIGHEST
    offx = jnp.dot(q, sowx_ref[...], preferred_element_type=jnp.float32) + sobx_ref[...]
    offy = jnp.dot(q, sowy_ref[...], preferred_element_type=jnp.float32) + soby_ref[...]
    logits = jnp.dot(q, aww_ref[...], preferred_element_type=jnp.float32) + awb_ref[...]
    val_ref[...] = jnp.dot(f_ref[...], vpw_ref[...], preferred_element_type=jnp.float32) + vpb_ref[...]
    rp = rp_ref[...]
    rpx = jnp.dot(rp, sx_ref[...], precision=hp, preferred_element_type=jnp.float32)
    rpy = jnp.dot(rp, sy_ref[...], precision=hp, preferred_element_type=jnp.float32)
    m = jnp.max(logits, axis=1, keepdims=True)
    e = jnp.exp(logits - m)
    s = jnp.dot(e, seg_ref[...], precision=hp, preferred_element_type=jnp.float32)
    aw = e / s
    wl = wl_ref[...]
    x = (rpx + offx / wl) * wl - 0.5
    y = (rpy + offy / wl) * wl - 0.5
    x0 = jnp.floor(x)
    fx = x - x0
    y0 = jnp.floor(y)
    fy = y - y0
    xs = jnp.clip(x0, 0.0, wl - 2.0)
    ys = jnp.clip(y0, 0.0, wl - 2.0)
    dx = x0 - xs
    dy = y0 - ys
    z = jnp.zeros_like(x)
    wxlo = jnp.where(dx == 0.0, 1.0 - fx, jnp.where(dx == -1.0, fx, z))
    wxhi = jnp.where(dx == 1.0, 1.0 - fx, jnp.where(dx == 0.0, fx, z))
    wylo = jnp.where(dy == 0.0, 1.0 - fy, jnp.where(dy == -1.0, fy, z)) * aw
    wyhi = jnp.where(dy == 1.0, 1.0 - fy, jnp.where(dy == 0.0, fy, z)) * aw
    w4_ref[:, 0:128] = wylo * wxlo
    w4_ref[:, 128:256] = wylo * wxhi
    w4_ref[:, 256:384] = wyhi * wxlo
    w4_ref[:, 384:512] = wyhi * wxhi
    bsel = (pl.program_id(0) >= _GRID // 2).astype(jnp.float32)
    gidx = (bsel * LIN + ls_ref[...] + ys * wl + xs) * N_HEADS + hh_ref[...]
    idx_ref[...] = gidx.astype(jnp.int32)


def _prep_tc(q2, f2, rp, sowx, sowy, aww, vpw, sobx, soby, awb, vpb):
    n = q2.shape[0]
    blk = _TCBLK
    wl, ls, hh, sx, sy, seg = _lane_tables()
    rep = lambda shp: pl.BlockSpec(shp, lambda i: tuple(0 for _ in shp))
    return pl.pallas_call(
        _prep_body,
        grid=(n // blk,),
        in_specs=[
            pl.BlockSpec((blk, D_MODEL), lambda i: (i, 0)),
            pl.BlockSpec((blk, D_MODEL), lambda i: (i, 0)),
            pl.BlockSpec((blk, 16), lambda i: (i, 0)),
            rep((D_MODEL, 128)), rep((D_MODEL, 128)), rep((D_MODEL, 128)),
            rep((D_MODEL, 256)),
            rep((128,)), rep((128,)), rep((128,)), rep((256,)),
            rep((1, 128)), rep((1, 128)), rep((1, 128)),
            rep((16, 128)), rep((16, 128)), rep((128, 128)),
        ],
        out_specs=[
            pl.BlockSpec((blk, 128), lambda i: (i, 0)),
            pl.BlockSpec((blk, 512), lambda i: (i, 0)),
            pl.BlockSpec((blk, 256), lambda i: (i, 0)),
        ],
        out_shape=[
            jax.ShapeDtypeStruct((n, 128), jnp.int32),
            jax.ShapeDtypeStruct((n, 512), jnp.float32),
            jax.ShapeDtypeStruct((n, 256), jnp.float32),
        ],
    )(q2, f2, rp, sowx, sowy, aww, vpw, sobx, soby, awb, vpb,
      wl, ls, hh, sx, sy, seg)


def _proj_body(x_ref, w_ref, b_ref, o_ref):
    o_ref[...] = jnp.dot(x_ref[...], w_ref[...], preferred_element_type=jnp.float32) + b_ref[...]


def _proj(x, w_t, b):
    n, k = x.shape
    m = w_t.shape[1]
    blk = _TCBLK
    return pl.pallas_call(
        _proj_body,
        grid=(n // blk,),
        in_specs=[
            pl.BlockSpec((blk, k), lambda i: (i, 0)),
            pl.BlockSpec((k, m), lambda i: (0, 0)),
            pl.BlockSpec((m,), lambda i: (0,)),
        ],
        out_specs=pl.BlockSpec((blk, m), lambda i: (i, 0)),
        out_shape=jax.ShapeDtypeStruct((n, m), jnp.float32),
    )(x, w_t, b)


def _sc_body(table_h, idx_h, w_h, out_h,
             out_v, idx_s0, idx_s1, w_s0, w_s1, rows_0, rows_1,
             sem_t0, sem_t1, sem_g0, sem_g1):
    iota = lax.iota(jnp.int32, 16)
    zero16 = jnp.full((16,), 0.0, jnp.float32)

    wid = lax.axis_index("c") * NS + lax.axis_index("s")
    q0 = wid * QPW

    def stage_pair(m, idx_s, w_s, sem):
        qf = q0 + m * 2 * QB
        pltpu.async_copy(idx_h.at[pl.ds(qf, 2 * QB)], idx_s, sem)
        pltpu.async_copy(w_h.at[pl.ds(qf * 512, 2 * QB * 512)], w_s, sem)

    def wait_pair(m, idx_s, w_s, sem):
        qf = q0 + m * 2 * QB
        pltpu.make_async_copy(idx_h.at[pl.ds(qf, 2 * QB)], idx_s, sem).wait()
        pltpu.make_async_copy(w_h.at[pl.ds(qf * 512, 2 * QB * 512)], w_s, sem).wait()

    def fire(idx_s, half, rows_r, sem):
        for qq in range(QB):
            pltpu.async_copy(table_h.at[idx_s.at[half * QB + qq]],
                             rows_r.at[pl.ds(qq * SPQ, SPQ)], sem)

    def drain(idx_s, half, rows_r, sem):
        for qq in range(QB):
            pltpu.make_async_copy(table_h.at[idx_s.at[half * QB + qq]],
                                  rows_r.at[pl.ds(qq * SPQ, SPQ)], sem).wait()

    def accum(k, w_s, half, rows_r):
        qf = q0 + k * QB
        for qq in range(QB):
            for h in range(N_HEADS):
                rbase = qq * SPQ + h * 16
                wbase = (half * QB + qq) * 512 + h * 16
                wsl = [w_s[pl.ds(wbase + slot * 128, 16)] for slot in range(4)]

                def body(t, carry, rbase=rbase, wsl=wsl):
                    acc = list(carry)
                    s = rbase + t
                    tv = jnp.full((16,), 0, jnp.int32) + t
                    for slot in range(4):
                        w = wsl[slot].at[tv].get(mode="promise_in_bounds")
                        acc[slot] = acc[slot] + w * rows_r[s, pl.ds(slot * 32, 16)]
                        acc[slot + 4] = acc[slot + 4] + w * rows_r[s, pl.ds(slot * 32 + 16, 16)]
                    return tuple(acc)

                acc = lax.fori_loop(0, 16, body, (zero16,) * 8)
                lo = (acc[0] + acc[1]) + (acc[2] + acc[3])
                hi = (acc[4] + acc[5]) + (acc[6] + acc[7])
                out_v[qq, pl.ds(h * 32, 16)] = lo
                out_v[qq, pl.ds(h * 32 + 16, 16)] = hi
        pltpu.sync_copy(out_v, out_h.at[pl.ds(qf, QB)])

    S0 = (idx_s0, w_s0, sem_t0)
    S1 = (idx_s1, w_s1, sem_t1)

    def pairstep(m, sp, sn):
        stage_pair(m + 1, *sn)
        fire(sp[0], 1, rows_1, sem_g1)
        drain(sp[0], 0, rows_0, sem_g0)
        accum(2 * m, sp[1], 0, rows_0)
        wait_pair(m + 1, *sn)
        fire(sn[0], 0, rows_0, sem_g0)
        drain(sp[0], 1, rows_1, sem_g1)
        accum(2 * m + 1, sp[1], 1, rows_1)

    stage_pair(0, *S0)
    wait_pair(0, *S0)
    fire(idx_s0, 0, rows_0, sem_g0)

    NPAIR = NBLK // 2

    def pipe(j, _):
        pairstep(2 * j, S0, S1)
        pairstep(2 * j + 1, S1, S0)
        return 0

    lax.fori_loop(0, (NPAIR - 1) // 2, pipe, 0)
    fire(idx_s0, 1, rows_1, sem_g1)
    drain(idx_s0, 0, rows_0, sem_g0)
    accum(NBLK - 2, w_s0, 0, rows_0)
    drain(idx_s0, 1, rows_1, sem_g1)
    accum(NBLK - 1, w_s0, 1, rows_1)


@jax.jit
def _sc_gather(table, idx, w4, ):
    mesh = plsc.VectorSubcoreMesh(core_axis_name="c", subcore_axis_name="s",
                                  num_cores=NC, num_subcores=NS)
    f = functools.partial(
        pl.kernel,
        out_type=jax.ShapeDtypeStruct((B * LQ, D_MODEL), jnp.float32),
        mesh=mesh,
        scratch_types=[
            pltpu.VMEM((QB, 256), jnp.float32),
            pltpu.VMEM((2 * QB, 128), jnp.int32),
            pltpu.VMEM((2 * QB, 128), jnp.int32),
            pltpu.VMEM((2 * QB * 512,), jnp.float32),
            pltpu.VMEM((2 * QB * 512,), jnp.float32),
            pltpu.VMEM((ROWS_PER_BLK, 128), jnp.float32),
            pltpu.VMEM((ROWS_PER_BLK, 128), jnp.float32),
            pltpu.SemaphoreType.DMA,
            pltpu.SemaphoreType.DMA,
            pltpu.SemaphoreType.DMA,
            pltpu.SemaphoreType.DMA,
        ],
        compiler_params=pltpu.CompilerParams(needs_layout_passes=False),
    )(_sc_body)
    return f(table, idx, w4)


def _build_patch_table(val):
    v = val.reshape(B, LIN, D_MODEL)
    parts = []
    for (H, W), s in zip(SPATIAL, LVL_START):
        vl = v[:, s:s + H * W]
        z = lambda n: jnp.zeros((B, n, D_MODEL), jnp.float32)
        v1 = jnp.concatenate([vl[:, 1:], z(1)], 1)
        vW = jnp.concatenate([vl[:, W:], z(W)], 1)
        vW1 = jnp.concatenate([vl[:, W + 1:], z(W + 1)], 1)
        parts.append(jnp.stack([vl, v1, vW, vW1], 2))
    patch = jnp.concatenate(parts, 1)
    patch = patch.reshape(B, LIN, 4, N_HEADS, DH).transpose(0, 1, 3, 2, 4)
    return patch.reshape(B * LIN * N_HEADS, 4 * DH)


def kernel(query, reference_points, input_flatten, input_spatial_shapes,
           input_level_start_index, so_w, so_b, aw_w, aw_b, vp_w, vp_b,
           op_w, op_b):
    q2 = query.reshape(B * LQ, D_MODEL)
    f2 = input_flatten.reshape(B * LIN, D_MODEL)
    rp = reference_points.reshape(B * LQ, 8)
    rp = jnp.concatenate([rp, jnp.zeros((B * LQ, 8), jnp.float32)], axis=1)
    sowt = so_w.T
    idx, w4, val = _prep_tc(q2, f2, rp, sowt[:, 0::2], sowt[:, 1::2], aw_w.T,
                            vp_w.T, so_b[0::2], so_b[1::2], aw_b, vp_b)
    table = _build_patch_table(val)
    out_core = _sc_gather(table, idx, w4.reshape(-1))
    o = _proj(out_core, op_w.T, op_b)
    return o.reshape(B, LQ, D_MODEL)


# speedup vs baseline: 1.2828x; 1.0826x over previous
import functools

import jax
import jax.numpy as jnp
import numpy as np
from jax import lax
from jax.experimental import pallas as pl
from jax.experimental.pallas import tpu as pltpu
from jax.experimental.pallas import tpu_sc as plsc

D_MODEL = 256
N_LEVELS = 4
N_HEADS = 8
N_POINTS = 4
DH = D_MODEL // N_HEADS
SPATIAL = [(64, 64), (32, 32), (16, 16), (8, 8)]
LVL_START = [0, 4096, 5120, 5376]
LIN = 5440
B = 2
LQ = LIN

NC = 2
NS = 16
NW = NC * NS
QPW = (B * LQ) // NW
QB = 2
NBLK = QPW // QB
SPQ = N_HEADS * N_LEVELS * N_POINTS
ROWS_PER_BLK = QB * SPQ

_TCBLK = 1088
_GRID = (B * LQ) // _TCBLK


def _lane_tables():
    j = np.arange(128)
    l = (j >> 2) & 3
    h = j >> 4
    w = np.array([64, 32, 16, 8], np.float32)[l]
    ls = np.array(LVL_START, np.float32)[l]
    sx = np.zeros((16, 128), np.float32)
    sy = np.zeros((16, 128), np.float32)
    sx[2 * l, j] = 1.0
    sy[2 * l + 1, j] = 1.0
    seg = np.zeros((128, 128), np.float32)
    seg[(j[:, None] >> 4) == (j[None, :] >> 4)] = 1.0
    return (jnp.asarray(w)[None], jnp.asarray(ls)[None],
            jnp.asarray(h.astype(np.float32))[None],
            jnp.asarray(sx), jnp.asarray(sy), jnp.asarray(seg))


def _prep_body(q_ref, f_ref, rp_ref, sowx_ref, sowy_ref, aww_ref, vpw_ref,
               sobx_ref, soby_ref, awb_ref, vpb_ref,
               wl_ref, ls_ref, hh_ref, sx_ref, sy_ref, seg_ref,
               idx_ref, w4_ref, val_ref):
    q = q_ref[...]
    hp = jax.lax.Precision.HIGHEST
    offx = jnp.dot(q, sowx_ref[...], preferred_element_type=jnp.float32) + sobx_ref[...]
    offy = jnp.dot(q, sowy_ref[...], preferred_element_type=jnp.float32) + soby_ref[...]
    logits = jnp.dot(q, aww_ref[...], preferred_element_type=jnp.float32) + awb_ref[...]
    val_ref[...] = jnp.dot(f_ref[...], vpw_ref[...], preferred_element_type=jnp.float32) + vpb_ref[...]
    rp = rp_ref[...]
    rpx = jnp.dot(rp, sx_ref[...], precision=hp, preferred_element_type=jnp.float32)
    rpy = jnp.dot(rp, sy_ref[...], precision=hp, preferred_element_type=jnp.float32)
    m = jnp.max(logits, axis=1, keepdims=True)
    e = jnp.exp(logits - m)
    s = jnp.dot(e, seg_ref[...], precision=hp, preferred_element_type=jnp.float32)
    aw = e / s
    wl = wl_ref[...]
    x = (rpx + offx / wl) * wl - 0.5
    y = (rpy + offy / wl) * wl - 0.5
    x0 = jnp.floor(x)
    fx = x - x0
    y0 = jnp.floor(y)
    fy = y - y0
    xs = jnp.clip(x0, 0.0, wl - 2.0)
    ys = jnp.clip(y0, 0.0, wl - 2.0)
    dx = x0 - xs
    dy = y0 - ys
    z = jnp.zeros_like(x)
    wxlo = jnp.where(dx == 0.0, 1.0 - fx, jnp.where(dx == -1.0, fx, z))
    wxhi = jnp.where(dx == 1.0, 1.0 - fx, jnp.where(dx == 0.0, fx, z))
    wylo = jnp.where(dy == 0.0, 1.0 - fy, jnp.where(dy == -1.0, fy, z)) * aw
    wyhi = jnp.where(dy == 1.0, 1.0 - fy, jnp.where(dy == 0.0, fy, z)) * aw
    w4_ref[:, 0:128] = wylo * wxlo
    w4_ref[:, 128:256] = wylo * wxhi
    w4_ref[:, 256:384] = wyhi * wxlo
    w4_ref[:, 384:512] = wyhi * wxhi
    bsel = (pl.program_id(0) >= _GRID // 2).astype(jnp.float32)
    gidx = (bsel * LIN + ls_ref[...] + ys * wl + xs) * N_HEADS + hh_ref[...]
    idx_ref[...] = gidx.astype(jnp.int32)


def _prep_tc(q2, f2, rp, sowx, sowy, aww, vpw, sobx, soby, awb, vpb):
    n = q2.shape[0]
    blk = _TCBLK
    wl, ls, hh, sx, sy, seg = _lane_tables()
    rep = lambda shp: pl.BlockSpec(shp, lambda i: tuple(0 for _ in shp))
    return pl.pallas_call(
        _prep_body,
        grid=(n // blk,),
        in_specs=[
            pl.BlockSpec((blk, D_MODEL), lambda i: (i, 0)),
            pl.BlockSpec((blk, D_MODEL), lambda i: (i, 0)),
            pl.BlockSpec((blk, 16), lambda i: (i, 0)),
            rep((D_MODEL, 128)), rep((D_MODEL, 128)), rep((D_MODEL, 128)),
            rep((D_MODEL, 256)),
            rep((128,)), rep((128,)), rep((128,)), rep((256,)),
            rep((1, 128)), rep((1, 128)), rep((1, 128)),
            rep((16, 128)), rep((16, 128)), rep((128, 128)),
        ],
        out_specs=[
            pl.BlockSpec((blk, 128), lambda i: (i, 0)),
            pl.BlockSpec((blk, 512), lambda i: (i, 0)),
            pl.BlockSpec((blk, 256), lambda i: (i, 0)),
        ],
        out_shape=[
            jax.ShapeDtypeStruct((n, 128), jnp.int32),
            jax.ShapeDtypeStruct((n, 512), jnp.float32),
            jax.ShapeDtypeStruct((n, 256), jnp.float32),
        ],
    )(q2, f2, rp, sowx, sowy, aww, vpw, sobx, soby, awb, vpb,
      wl, ls, hh, sx, sy, seg)


def _proj_body(x_ref, w_ref, b_ref, o_ref):
    o_ref[...] = jnp.dot(x_ref[...], w_ref[...], preferred_element_type=jnp.float32) + b_ref[...]


def _proj(x, w_t, b):
    n, k = x.shape
    m = w_t.shape[1]
    blk = _TCBLK
    return pl.pallas_call(
        _proj_body,
        grid=(n // blk,),
        in_specs=[
            pl.BlockSpec((blk, k), lambda i: (i, 0)),
            pl.BlockSpec((k, m), lambda i: (0, 0)),
            pl.BlockSpec((m,), lambda i: (0,)),
        ],
        out_specs=pl.BlockSpec((blk, m), lambda i: (i, 0)),
        out_shape=jax.ShapeDtypeStruct((n, m), jnp.float32),
    )(x, w_t, b)


def _sc_body(table_h, idx_h, w_h, out_h,
             out_v, idx_s0, idx_s1, w_s0, w_s1, rows_0, rows_1,
             sem_t0, sem_t1, sem_g0, sem_g1):
    iota = lax.iota(jnp.int32, 16)
    zero16 = jnp.full((16,), 0.0, jnp.float32)

    wid = lax.axis_index("c") * NS + lax.axis_index("s")
    q0 = wid * QPW

    def stage_pair(m, idx_s, w_s, sem):
        qf = q0 + m * 2 * QB
        pltpu.async_copy(idx_h.at[pl.ds(qf, 2 * QB)], idx_s, sem)
        pltpu.async_copy(w_h.at[pl.ds(qf, 2 * QB)], w_s, sem)

    def wait_pair(m, idx_s, w_s, sem):
        qf = q0 + m * 2 * QB
        pltpu.make_async_copy(idx_h.at[pl.ds(qf, 2 * QB)], idx_s, sem).wait()
        pltpu.make_async_copy(w_h.at[pl.ds(qf, 2 * QB)], w_s, sem).wait()

    def fire(idx_s, half, rows_r, sem):
        for qq in range(QB):
            pltpu.async_copy(table_h.at[idx_s.at[half * QB + qq]],
                             rows_r.at[pl.ds(qq * SPQ, SPQ)], sem)

    def drain(idx_s, half, rows_r, sem):
        for qq in range(QB):
            pltpu.make_async_copy(table_h.at[idx_s.at[half * QB + qq]],
                                  rows_r.at[pl.ds(qq * SPQ, SPQ)], sem).wait()

    def accum(k, w_s, half, rows_r):
        qf = q0 + k * QB
        for qq in range(QB):
            for h in range(N_HEADS):
                rbase = qq * SPQ + h * 16
                wsl = [w_s[half * QB + qq, pl.ds(slot * 128 + h * 16, 16)]
                       for slot in range(4)]

                def body(t, carry, rbase=rbase, wsl=wsl):
                    acc = list(carry)
                    s = rbase + t
                    tv = jnp.full((16,), 0, jnp.int32) + t
                    for slot in range(4):
                        w = wsl[slot].at[tv].get(mode="promise_in_bounds")
                        acc[slot] = acc[slot] + w * rows_r[s, pl.ds(slot * 32, 16)]
                        acc[slot + 4] = acc[slot + 4] + w * rows_r[s, pl.ds(slot * 32 + 16, 16)]
                    return tuple(acc)

                acc = lax.fori_loop(0, 16, body, (zero16,) * 8)
                lo = (acc[0] + acc[1]) + (acc[2] + acc[3])
                hi = (acc[4] + acc[5]) + (acc[6] + acc[7])
                out_v[qq, pl.ds(h * 32, 16)] = lo
                out_v[qq, pl.ds(h * 32 + 16, 16)] = hi
        pltpu.sync_copy(out_v, out_h.at[pl.ds(qf, QB)])

    S0 = (idx_s0, w_s0, sem_t0)
    S1 = (idx_s1, w_s1, sem_t1)

    def pairstep(m, sp, sn):
        stage_pair(m + 1, *sn)
        fire(sp[0], 1, rows_1, sem_g1)
        drain(sp[0], 0, rows_0, sem_g0)
        accum(2 * m, sp[1], 0, rows_0)
        wait_pair(m + 1, *sn)
        fire(sn[0], 0, rows_0, sem_g0)
        drain(sp[0], 1, rows_1, sem_g1)
        accum(2 * m + 1, sp[1], 1, rows_1)

    stage_pair(0, *S0)
    wait_pair(0, *S0)
    fire(idx_s0, 0, rows_0, sem_g0)

    NPAIR = NBLK // 2

    def pipe(j, _):
        pairstep(2 * j, S0, S1)
        pairstep(2 * j + 1, S1, S0)
        return 0

    lax.fori_loop(0, (NPAIR - 1) // 2, pipe, 0)
    fire(idx_s0, 1, rows_1, sem_g1)
    drain(idx_s0, 0, rows_0, sem_g0)
    accum(NBLK - 2, w_s0, 0, rows_0)
    drain(idx_s0, 1, rows_1, sem_g1)
    accum(NBLK - 1, w_s0, 1, rows_1)


@jax.jit
def _sc_gather(table, idx, w4, ):
    mesh = plsc.VectorSubcoreMesh(core_axis_name="c", subcore_axis_name="s",
                                  num_cores=NC, num_subcores=NS)
    f = functools.partial(
        pl.kernel,
        out_type=jax.ShapeDtypeStruct((B * LQ, D_MODEL), jnp.float32),
        mesh=mesh,
        scratch_types=[
            pltpu.VMEM((QB, 256), jnp.float32),
            pltpu.VMEM((2 * QB, 128), jnp.int32),
            pltpu.VMEM((2 * QB, 128), jnp.int32),
            pltpu.VMEM((2 * QB, 512), jnp.float32),
            pltpu.VMEM((2 * QB, 512), jnp.float32),
            pltpu.VMEM((ROWS_PER_BLK, 128), jnp.float32),
            pltpu.VMEM((ROWS_PER_BLK, 128), jnp.float32),
            pltpu.SemaphoreType.DMA,
            pltpu.SemaphoreType.DMA,
            pltpu.SemaphoreType.DMA,
            pltpu.SemaphoreType.DMA,
        ],
        compiler_params=pltpu.CompilerParams(needs_layout_passes=False),
    )(_sc_body)
    return f(table, idx, w4)


def _build_patch_table(val):
    v = val.reshape(B, LIN, D_MODEL)
    parts = []
    for (H, W), s in zip(SPATIAL, LVL_START):
        vl = v[:, s:s + H * W]
        z = lambda n: jnp.zeros((B, n, D_MODEL), jnp.float32)
        v1 = jnp.concatenate([vl[:, 1:], z(1)], 1)
        vW = jnp.concatenate([vl[:, W:], z(W)], 1)
        vW1 = jnp.concatenate([vl[:, W + 1:], z(W + 1)], 1)
        parts.append(jnp.stack([vl, v1, vW, vW1], 2))
    patch = jnp.concatenate(parts, 1)
    patch = patch.reshape(B, LIN, 4, N_HEADS, DH).transpose(0, 1, 3, 2, 4)
    return patch.reshape(B * LIN * N_HEADS, 4 * DH)


def kernel(query, reference_points, input_flatten, input_spatial_shapes,
           input_level_start_index, so_w, so_b, aw_w, aw_b, vp_w, vp_b,
           op_w, op_b):
    q2 = query.reshape(B * LQ, D_MODEL)
    f2 = input_flatten.reshape(B * LIN, D_MODEL)
    rp = reference_points.reshape(B * LQ, 8)
    rp = jnp.concatenate([rp, jnp.zeros((B * LQ, 8), jnp.float32)], axis=1)
    sowt = so_w.T
    idx, w4, val = _prep_tc(q2, f2, rp, sowt[:, 0::2], sowt[:, 1::2], aw_w.T,
                            vp_w.T, so_b[0::2], so_b[1::2], aw_b, vp_b)
    table = _build_patch_table(val)
    out_core = _sc_gather(table, idx, w4)
    o = _proj(out_core, op_w.T, op_b)
    return o.reshape(B, LQ, D_MODEL)
